# baseline (speedup 1.0000x reference)
import functools

import jax
import jax.numpy as jnp
from jax import lax
from jax.experimental import pallas as pl
from jax.experimental.pallas import tpu as pltpu
from jax.experimental.pallas import tpu_sc as plsc

B = 4096
T = 77
D = 768
V = 49408

NC = 2
NS = 16
NW = NC * NS
BPW = B // NW
NBUF = 3
CS = (48, 48, 32)
OFF = (0, 48, 96)
HS = tuple(c // 2 for c in CS)
NVREG = D // 16


def _make_embed_kernel():
    mesh = plsc.VectorSubcoreMesh(core_axis_name="c", subcore_axis_name="s")

    scratch = (
        [pltpu.VMEM((CS[s],), jnp.int32) for s in range(NBUF)]
        + [pltpu.VMEM((D,), jnp.float32) for s in range(NBUF)]
        + [pltpu.VMEM((CS[s], D), jnp.float32) for s in range(NBUF)]
        + [pltpu.SemaphoreType.DMA for _ in range(4 * NBUF)]
    )

    @functools.partial(
        pl.kernel,
        out_type=jax.ShapeDtypeStruct((T * B, D), jnp.float32),
        mesh=mesh,
        scratch_types=scratch,
    )
    def embed(ids_hbm, table_hbm, pos_hbm, out_hbm, *scr):
        idx = scr[0:NBUF]
        pos = scr[NBUF:2 * NBUF]
        rows = scr[2 * NBUF:3 * NBUF]
        gasems = scr[3 * NBUF:4 * NBUF]
        gbsems = scr[4 * NBUF:5 * NBUF]
        osems = scr[5 * NBUF:6 * NBUF]
        ssems = scr[6 * NBUF:7 * NBUF]

        wid = lax.axis_index("s") * NC + lax.axis_index("c")
        b_base = wid * BPW

        def start_gather(s):
            h = HS[s]
            pltpu.async_copy(table_hbm.at[idx[s].at[pl.ds(0, h)]],
                             rows[s].at[pl.ds(0, h)], gasems[s])
            pltpu.async_copy(table_hbm.at[idx[s].at[pl.ds(h, h)]],
                             rows[s].at[pl.ds(h, h)], gbsems[s])

        def stage(p, s):
            b0 = b_base + OFF[s]
            return (
                pltpu.make_async_copy(ids_hbm.at[pl.ds(p * B + b0, CS[s])],
                                      idx[s], ssems[s]),
                pltpu.make_async_copy(pos_hbm.at[pl.ds(p * D, D)],
                                      pos[s], ssems[s]),
            )

        def out_half(p, s, q):
            h = HS[s]
            b0 = b_base + OFF[s]
            return pltpu.make_async_copy(
                rows[s].at[pl.ds(q * h, h)],
                out_hbm.at[pl.ds(p * B + b0 + q * h, h)], osems[s])

        def restart(ap, s):
            for d in stage(ap, s):
                d.wait()
            out_half(ap - 1, s, 0).wait()
            out_half(ap - 1, s, 1).wait()
            start_gather(s)

        def process(p, slot):
            idxb, posb, rowsb = idx[slot], pos[slot], rows[slot]
            h = HS[slot]
            b0 = b_base + OFF[slot]

            if slot == 0:
                @pl.when(p >= 1)
                def _():
                    restart(p, 2)
                @pl.when(p == 0)
                def _():
                    start_gather(2)
            else:
                @pl.when(p + 1 < T)
                def _():
                    restart(p + 1, slot - 1)

            def row_body(b, pv):
                for k in range(NVREG):
                    plsc.addupdate(rowsb.at[b, pl.ds(k * 16, 16)], pv[k])
                return pv

            pltpu.make_async_copy(table_hbm.at[idxb.at[pl.ds(0, h)]],
                                  rowsb.at[pl.ds(0, h)], gasems[slot]).wait()
            pvs = tuple(posb[pl.ds(k * 16, 16)] for k in range(NVREG))
            pvs = lax.fori_loop(0, h, row_body, pvs, unroll=2)
            out_half(p, slot, 0).start()
            pltpu.make_async_copy(table_hbm.at[idxb.at[pl.ds(h, h)]],
                                  rowsb.at[pl.ds(h, h)], gbsems[slot]).wait()
            @pl.when(p + 1 < T)
            def _():
                for d in stage(p + 1, slot):
                    d.start()
            lax.fori_loop(h, CS[slot], row_body, pvs, unroll=2)
            out_half(p, slot, 1).start()

        for s in range(NBUF):
            b0 = b_base + OFF[s]
            pltpu.sync_copy(ids_hbm.at[pl.ds(b0, CS[s])], idx[s])
            pltpu.sync_copy(pos_hbm.at[pl.ds(0, D)], pos[s])
        start_gather(0)
        start_gather(1)

        def trip(p, acc):
            for slot in range(NBUF):
                process(p, slot)
            return acc

        lax.fori_loop(0, T, trip, 0)

        for s in range(NBUF):
            out_half(T - 1, s, 0).wait()
            out_half(T - 1, s, 1).wait()

    return embed


_embed = _make_embed_kernel()


@jax.jit
def kernel(input_ids, table, pos_table):
    ids_t = input_ids.astype(jnp.int32).T.reshape(-1)
    pos_flat = pos_table.reshape(-1)
    out_flat = _embed(ids_t, table, pos_flat)
    return out_flat.reshape(T, B, D).transpose(1, 0, 2)


# speedup vs baseline: 1.0342x; 1.0342x over previous
import functools

import jax
import jax.numpy as jnp
from jax import lax
from jax.experimental import pallas as pl
from jax.experimental.pallas import tpu as pltpu
from jax.experimental.pallas import tpu_sc as plsc

B = 4096
T = 77
D = 768
V = 49408

NC = 2
NS = 16
NW = NC * NS
BPW = B // NW
NBUF = 2
C = BPW // NBUF
H = C // 2
NVREG = D // 16


def _make_embed_kernel():
    mesh = plsc.VectorSubcoreMesh(core_axis_name="c", subcore_axis_name="s")

    scratch = (
        [pltpu.VMEM((C,), jnp.int32) for _ in range(NBUF)]
        + [pltpu.VMEM((D,), jnp.float32) for _ in range(NBUF)]
        + [pltpu.VMEM((C, D), jnp.float32) for _ in range(NBUF)]
        + [pltpu.SemaphoreType.DMA for _ in range(4 * NBUF)]
    )

    @functools.partial(
        pl.kernel,
        out_type=jax.ShapeDtypeStruct((T * B, D), jnp.float32),
        mesh=mesh,
        scratch_types=scratch,
    )
    def embed(ids_hbm, table_hbm, pos_hbm, out_hbm, *scr):
        idx = scr[0:NBUF]
        pos = scr[NBUF:2 * NBUF]
        rows = scr[2 * NBUF:3 * NBUF]
        gasems = scr[3 * NBUF:4 * NBUF]
        gbsems = scr[4 * NBUF:5 * NBUF]
        osems = scr[5 * NBUF:6 * NBUF]
        ssems = scr[6 * NBUF:7 * NBUF]

        wid = lax.axis_index("s") * NC + lax.axis_index("c")
        b_base = wid * BPW

        def start_gather(p, slot):
            idxb, rowsb = idx[slot], rows[slot]
            pltpu.async_copy(table_hbm.at[idxb.at[pl.ds(0, H)]],
                             rowsb.at[pl.ds(0, H)], gasems[slot])
            pltpu.async_copy(table_hbm.at[idxb.at[pl.ds(H, H)]],
                             rowsb.at[pl.ds(H, H)], gbsems[slot])

        def fetch(p, slot):
            b0 = b_base + slot * C
            pltpu.sync_copy(ids_hbm.at[pl.ds(p * B + b0, C)], idx[slot])
            pltpu.sync_copy(pos_hbm.at[pl.ds(p * D, D)], pos[slot])
            start_gather(p, slot)

        def process(p, slot):
            idxb, posb, rowsb = idx[slot], pos[slot], rows[slot]
            osem, ssem = osems[slot], ssems[slot]
            b0 = b_base + slot * C

            def row_body(b, pv):
                for k in range(NVREG):
                    plsc.addupdate(rowsb.at[b, pl.ds(k * 16, 16)], pv[k])
                return pv

            def out_half(q):
                return pltpu.make_async_copy(
                    rowsb.at[pl.ds(q * H, H)],
                    out_hbm.at[pl.ds(p * B + b0 + q * H, H)], osem)

            pltpu.make_async_copy(table_hbm.at[idxb.at[pl.ds(0, H)]],
                                  rowsb.at[pl.ds(0, H)], gasems[slot]).wait()
            pvs = tuple(posb[pl.ds(k * 16, 16)] for k in range(NVREG))
            pvs = lax.fori_loop(0, H, row_body, pvs, unroll=2)
            out_half(0).start()
            pltpu.make_async_copy(table_hbm.at[idxb.at[pl.ds(H, H)]],
                                  rowsb.at[pl.ds(H, H)], gbsems[slot]).wait()
            @pl.when(p + 1 < T)
            def _():
                pltpu.async_copy(ids_hbm.at[pl.ds((p + 1) * B + b0, C)],
                                 idxb, ssem)
                pltpu.async_copy(pos_hbm.at[pl.ds((p + 1) * D, D)], posb, ssem)
            lax.fori_loop(H, C, row_body, pvs, unroll=2)
            out_half(1).start()

            @pl.when(p + 1 < T)
            def _():
                pltpu.make_async_copy(ids_hbm.at[pl.ds((p + 1) * B + b0, C)],
                                      idxb, ssem).wait()
                pltpu.make_async_copy(pos_hbm.at[pl.ds((p + 1) * D, D)],
                                      posb, ssem).wait()
                out_half(0).wait()
                out_half(1).wait()
                start_gather(p + 1, slot)

        for slot in range(NBUF):
            fetch(0, slot)

        def trip(p, acc):
            for slot in range(NBUF):
                process(p, slot)
            return acc

        lax.fori_loop(0, T, trip, 0)

        for slot in range(NBUF):
            b0 = b_base + slot * C
            for q in range(2):
                pltpu.make_async_copy(
                    rows[slot].at[pl.ds(q * H, H)],
                    out_hbm.at[pl.ds((T - 1) * B + b0 + q * H, H)],
                    osems[slot]).wait()

    return embed


_embed = _make_embed_kernel()


@jax.jit
def kernel(input_ids, table, pos_table):
    ids_t = input_ids.astype(jnp.int32).T.reshape(-1)
    pos_flat = pos_table.reshape(-1)
    out_flat = _embed(ids_t, table, pos_flat)
    return out_flat.reshape(T, B, D).transpose(1, 0, 2)
